# baseline (speedup 1.0000x reference)
import dataclasses
import functools

import jax
import jax.numpy as jnp
import numpy as np
from jax import lax
from jax.experimental import pallas as pl
from jax.experimental.pallas import tpu as pltpu
from jax.experimental.pallas import tpu_sc as plsc

N_NODES = 10000
N_EDGES = 320000
D = 128
G = 128
NW = 32
GROUPS_PER_W = 84
E_PAD = NW * GROUPS_PER_W * G
SLOTS = 12
N_ITERS = GROUPS_PER_W // SLOTS
ROWS_PER_SUB = 624
ROWS_TAIL = N_NODES - 16 * ROWS_PER_SUB

_f32 = jnp.float32
_bf16 = jnp.bfloat16

PERM3 = np.empty((D,), dtype=np.int32)
for _j in range(64):
    _q, _u = _j // 16, _j % 16
    PERM3[_j] = 32 * _q + _u
    PERM3[64 + _j] = 32 * _q + 16 + _u


def _sc_agg_kernel(t_hbm, src_hbm, dst_hbm, norm_hbm, zeros_hbm, out_hbm,
                   src0, src1, src2, nrm0, nrm1, nrm2,
                   dst0, dst1, dst2, dst3,
                   r16a, r16b, rfa, rfb, agg_sp,
                   gs0, gs1, ss0, ss1, ds0, ds1, ds2, ds3,
                   is0, is1, is2):
    c = lax.axis_index("c")
    s = lax.axis_index("s")
    wid = s * 2 + c
    srcb = (src0, src1, src2)
    nrmb = (nrm0, nrm1, nrm2)
    dstb = (dst0, dst1, dst2, dst3)
    rows16 = (r16a, r16b)
    rowsf = (rfa, rfb)
    gsem = (gs0, gs1)
    ssem = (ss0, ss1)
    dsem = (ds0, ds1, ds2, ds3)
    isem = (is0, is1, is2)

    e_base = wid * (GROUPS_PER_W * G)

    def src_sl(g):
        return src_hbm.at[pl.ds(e_base + g * G, G)]

    def dst_sl(g):
        return dst_hbm.at[pl.ds(e_base + g * G, G)]

    def nrm_sl(g):
        return norm_hbm.at[pl.ds(e_base + g * G, G)]

    pltpu.sync_copy(src_sl(0), srcb[0])
    pltpu.sync_copy(nrm_sl(0), nrmb[0])
    pltpu.make_async_copy(src_sl(1), srcb[1], isem[1]).start()
    pltpu.make_async_copy(nrm_sl(1), nrmb[1], isem[1]).start()
    pltpu.make_async_copy(dst_sl(0), dstb[0], dsem[0]).start()
    pltpu.make_async_copy(dst_sl(1), dstb[1], dsem[1]).start()
    pltpu.make_async_copy(t_hbm.at[srcb[0]], rows16[0], gsem[0]).start()

    r0 = s * ROWS_PER_SUB
    pltpu.sync_copy(zeros_hbm.at[pl.ds(r0, ROWS_PER_SUB)],
                    agg_sp.at[pl.ds(r0, ROWS_PER_SUB)])

    @pl.when(s == 15)
    def _():
        pltpu.sync_copy(zeros_hbm.at[pl.ds(16 * ROWS_PER_SUB, ROWS_TAIL)],
                        agg_sp.at[pl.ds(16 * ROWS_PER_SUB, ROWS_TAIL)])

    plsc.subcore_barrier()

    def _scale(br, bi):
        src16 = rows16[br]
        dstf = rowsf[br]
        nbuf = nrmb[bi]

        def tile_body(r4, carry):
            for dr in range(4):
                r = r4 + dr
                nb = plsc.load_gather(nbuf, [lax.broadcast(r, (16,))])
                for q in range(4):
                    x16i = src16[r, pl.ds(16 * q, 16)]
                    x32 = plsc.bitcast(x16i, _bf16)
                    a, b = plsc.unpack(
                        x32, format=plsc.PackFormat.INTERLEAVED,
                        preferred_element_type=_f32)
                    dstf[r, pl.ds(32 * q, 16)] = a * nb
                    dstf[r, pl.ds(32 * q + 16, 16)] = b * nb
            return carry

        lax.fori_loop(0, G // 4, lambda t, cy: tile_body(t * 4, cy), 0)

    def iter_body(i, carry):
        for k in range(SLOTS):
            g = i * SLOTS + k
            br, bi, bd = k % 2, k % 3, k % 4

            pltpu.make_async_copy(t_hbm.at[srcb[bi]], rows16[br],
                                  gsem[br]).wait()

            @pl.when(g + 1 <= GROUPS_PER_W - 1)
            def _():
                bi1 = (k + 1) % 3
                pltpu.make_async_copy(src_sl(g + 1), srcb[bi1],
                                      isem[bi1]).wait()
                pltpu.make_async_copy(nrm_sl(g + 1), nrmb[bi1],
                                      isem[bi1]).wait()
                pltpu.make_async_copy(t_hbm.at[srcb[bi1]], rows16[1 - br],
                                      gsem[1 - br]).start()

            pltpu.make_async_copy(dst_sl(g), dstb[bd], dsem[bd]).wait()

            @pl.when(g >= 2)
            def _():
                pltpu.make_async_copy(rowsf[br], agg_sp.at[dstb[bd]],
                                      ssem[br]).wait()

            _scale(br, bi)
            pltpu.make_async_copy(rowsf[br], agg_sp.at[dstb[bd]],
                                  ssem[br]).start(add=True)

            @pl.when(g + 2 <= GROUPS_PER_W - 1)
            def _():
                bd2 = (k + 2) % 4
                bi2 = (k + 2) % 3
                pltpu.make_async_copy(dst_sl(g + 2), dstb[bd2],
                                      dsem[bd2]).start()
                pltpu.make_async_copy(src_sl(g + 2), srcb[bi2],
                                      isem[bi2]).start()
                pltpu.make_async_copy(nrm_sl(g + 2), nrmb[bi2],
                                      isem[bi2]).start()
        return carry

    lax.fori_loop(0, N_ITERS, iter_body, 0)

    for g in (GROUPS_PER_W - 2, GROUPS_PER_W - 1):
        pltpu.make_async_copy(rowsf[g % 2], agg_sp.at[dstb[g % 4]],
                              ssem[g % 2]).wait()

    plsc.subcore_barrier()
    pltpu.sync_copy(agg_sp.at[pl.ds(r0, ROWS_PER_SUB)],
                    out_hbm.at[c].at[pl.ds(r0, ROWS_PER_SUB)])

    @pl.when(s == 15)
    def _():
        pltpu.sync_copy(agg_sp.at[pl.ds(16 * ROWS_PER_SUB, ROWS_TAIL)],
                        out_hbm.at[c].at[pl.ds(16 * ROWS_PER_SUB, ROWS_TAIL)])


@jax.jit
def _sc_aggregate(t16, src_e, dst_e, norm_e, zeros):
    mesh = plsc.VectorSubcoreMesh(core_axis_name="c", subcore_axis_name="s")
    cp = pltpu.CompilerParams()
    if "needs_layout_passes" in pltpu.CompilerParams.__dataclass_fields__:
        cp = dataclasses.replace(cp, needs_layout_passes=False)
    cp = dataclasses.replace(cp, use_tc_tiling_on_sc=False)
    kfn = pl.kernel(
        _sc_agg_kernel,
        out_type=jax.ShapeDtypeStruct((2, N_NODES, D), _f32),
        mesh=mesh,
        scratch_types=(
            [pltpu.VMEM((G,), jnp.int32) for _ in range(3)]
            + [pltpu.VMEM((G,), _f32) for _ in range(3)]
            + [pltpu.VMEM((G,), jnp.int32) for _ in range(4)]
            + [pltpu.VMEM((G, D // 2), jnp.int32) for _ in range(2)]
            + [pltpu.VMEM((G, D), _f32) for _ in range(2)]
            + [pltpu.VMEM_SHARED((N_NODES, D), _f32)]
            + [pltpu.SemaphoreType.DMA for _ in range(11)]
        ),
        compiler_params=cp,
    )
    return kfn(t16, src_e, dst_e, norm_e, zeros)


def _pack_pairs(t):
    lo = jax.lax.bitcast_convert_type(t[:, :D // 2].astype(_bf16),
                                      jnp.uint16).astype(jnp.uint32)
    hi = jax.lax.bitcast_convert_type(t[:, D // 2:].astype(_bf16),
                                      jnp.uint16).astype(jnp.uint32)
    word = lo | (hi << 16)
    return jax.lax.bitcast_convert_type(word, jnp.int32)


def _mm_first_kernel(x_ref, w_ref, o_ref):
    o_ref[...] = _pack_pairs(jnp.dot(x_ref[...], w_ref[...],
                                     preferred_element_type=_f32))


def _mm_mid_kernel(p_ref, b_ref, w_ref, o_ref):
    h = jax.nn.relu(p_ref[0] + p_ref[1] + b_ref[...])
    o_ref[...] = _pack_pairs(jnp.dot(h, w_ref[...],
                                     preferred_element_type=_f32))


def _head_kernel(p_ref, b_ref, fcw_ref, fcb_ref, h_ref, y_ref):
    h = p_ref[0] + p_ref[1] + b_ref[...]
    h_ref[...] = h
    y = jnp.dot(h, fcw_ref[...], preferred_element_type=_f32)
    y = y + fcb_ref[...]
    m = jnp.max(y, axis=1, keepdims=True)
    e = jnp.exp(y - m)
    lse = jnp.log(jnp.sum(e, axis=1, keepdims=True)) + m
    y_ref[...] = y - lse


def _mm_first(x, w):
    return pl.pallas_call(
        _mm_first_kernel,
        out_shape=jax.ShapeDtypeStruct((x.shape[0], D // 2), jnp.int32),
    )(x, w)


def _mm_mid(p, b, w):
    return pl.pallas_call(
        _mm_mid_kernel,
        out_shape=jax.ShapeDtypeStruct((p.shape[1], D // 2), jnp.int32),
    )(p, b.reshape(1, -1), w)


def _head(p, b, fcW, fcb):
    return pl.pallas_call(
        _head_kernel,
        out_shape=(jax.ShapeDtypeStruct((p.shape[1], D), _f32),
                   jax.ShapeDtypeStruct((p.shape[1], fcW.shape[1]), _f32)),
    )(p, b.reshape(1, -1), fcW, fcb.reshape(1, -1))


def kernel(x, edge_index, edge_weight, W1, b1, W2, b2, W3, b3, fcW, fcb):
    pad = E_PAD - N_EDGES
    pad_idx = (jnp.arange(pad, dtype=jnp.int32) * 13) % N_NODES
    src_e = jnp.concatenate([edge_index[0], pad_idx])
    dst_e = jnp.concatenate([edge_index[1], pad_idx])
    norm_e = jnp.concatenate([edge_weight, jnp.zeros((pad,), _f32)])
    zeros = jnp.zeros((N_NODES, D), _f32)
    perm = jnp.asarray(PERM3)

    V1 = W1[:, perm]
    V2 = W2[:, perm]
    V3 = W3[:, perm]

    t1 = _mm_first(x, V1)
    p1 = _sc_aggregate(t1, src_e, dst_e, norm_e, zeros)
    t2 = _mm_mid(p1, b1, V2)
    p2 = _sc_aggregate(t2, src_e, dst_e, norm_e, zeros)
    t3 = _mm_mid(p2, b2, V3)
    p3 = _sc_aggregate(t3, src_e, dst_e, norm_e, zeros)
    h, y = _head(p3, b3, fcW, fcb)
    return h, y


# speedup vs baseline: 1.8216x; 1.8216x over previous
import dataclasses
import functools

import jax
import jax.numpy as jnp
from jax import lax
from jax.experimental import pallas as pl
from jax.experimental.pallas import tpu as pltpu
from jax.experimental.pallas import tpu_sc as plsc

N_NODES = 10000
N_EDGES = 320000
D = 128
G = 128
NW = 32
GROUPS_PER_W = 81
E_PAD = NW * GROUPS_PER_W * G
NBUF = 3
N_ITERS = GROUPS_PER_W // NBUF
ROWS_PER_SUB = 624
ROWS_TAIL = N_NODES - 16 * ROWS_PER_SUB

_f32 = jnp.float32


def _sc_agg_kernel(t_hbm, src_hbm, dst_hbm, norm_hbm, zeros_hbm, out_hbm,
                   src0, src1, src2, dst0, dst1, dst2, nrm0, nrm1, nrm2,
                   rows0, rows1, rows2, agg_sp,
                   gs0, gs1, gs2, ss0, ss1, ss2, ds0, ds1, ds2,
                   es0, es1, es2, en0, en1, en2):
    c = lax.axis_index("c")
    s = lax.axis_index("s")
    wid = s * 2 + c
    srcb = (src0, src1, src2)
    dstb = (dst0, dst1, dst2)
    nrmb = (nrm0, nrm1, nrm2)
    rows = (rows0, rows1, rows2)
    gsem = (gs0, gs1, gs2)
    ssem = (ss0, ss1, ss2)
    dsem = (ds0, ds1, ds2)
    isem_s = (es0, es1, es2)
    isem_n = (en0, en1, en2)

    e_base = wid * (GROUPS_PER_W * G)

    def src_sl(g):
        return src_hbm.at[pl.ds(e_base + g * G, G)]

    def dst_sl(g):
        return dst_hbm.at[pl.ds(e_base + g * G, G)]

    def nrm_sl(g):
        return norm_hbm.at[pl.ds(e_base + g * G, G)]

    for k in range(2):
        pltpu.sync_copy(src_sl(k), srcb[k])
        pltpu.make_async_copy(nrm_sl(k), nrmb[k], isem_n[k]).start()
        pltpu.make_async_copy(dst_sl(k), dstb[k], dsem[k]).start()
        pltpu.make_async_copy(t_hbm.at[srcb[k]], rows[k], gsem[k]).start()
    pltpu.make_async_copy(src_sl(2), srcb[2], isem_s[2]).start()

    r0 = s * ROWS_PER_SUB
    pltpu.sync_copy(zeros_hbm.at[pl.ds(r0, ROWS_PER_SUB)],
                    agg_sp.at[pl.ds(r0, ROWS_PER_SUB)])

    @pl.when(s == 15)
    def _():
        pltpu.sync_copy(zeros_hbm.at[pl.ds(16 * ROWS_PER_SUB, ROWS_TAIL)],
                        agg_sp.at[pl.ds(16 * ROWS_PER_SUB, ROWS_TAIL)])

    plsc.subcore_barrier()

    def _scale(buf, nbuf):
        def tile_body(r16, carry):
            for dr in range(4):
                r = r16 + dr
                nb = plsc.load_gather(nbuf, [lax.broadcast(r, (16,))])
                for j in range(8):
                    sl = pl.ds(j * 16, 16)
                    buf[r, sl] = buf[r, sl] * nb
            return carry

        lax.fori_loop(0, G // 4, lambda t, cy: tile_body(t * 4, cy), 0)

    def iter_body(i, carry):
        for k in range(NBUF):
            g = i * NBUF + k
            bp = (k + 2) % NBUF
            gp2 = g + 2
            have_next = gp2 <= GROUPS_PER_W - 1

            pltpu.make_async_copy(t_hbm.at[srcb[k]], rows[k],
                                  gsem[k]).wait()

            @pl.when(g + 3 <= GROUPS_PER_W - 1)
            def _():
                pltpu.make_async_copy(src_sl(g + 3), srcb[k],
                                      isem_s[k]).start()

            pltpu.make_async_copy(dst_sl(g), dstb[k], dsem[k]).wait()

            @pl.when(g >= 1)
            def _():
                pltpu.make_async_copy(rows[bp], agg_sp.at[dstb[bp]],
                                      ssem[bp]).wait()

            @pl.when(have_next)
            def _():
                pltpu.make_async_copy(dst_sl(gp2), dstb[bp],
                                      dsem[bp]).start()
                pltpu.make_async_copy(src_sl(gp2), srcb[bp],
                                      isem_s[bp]).wait()
                pltpu.make_async_copy(t_hbm.at[srcb[bp]], rows[bp],
                                      gsem[bp]).start()
                pltpu.make_async_copy(nrm_sl(gp2), nrmb[bp],
                                      isem_n[bp]).start()

            pltpu.make_async_copy(nrm_sl(g), nrmb[k], isem_n[k]).wait()
            _scale(rows[k], nrmb[k])
            pltpu.make_async_copy(rows[k], agg_sp.at[dstb[k]],
                                  ssem[k]).start(add=True)
        return carry

    lax.fori_loop(0, N_ITERS, iter_body, 0)

    kl = (GROUPS_PER_W - 1) % NBUF
    pltpu.make_async_copy(rows[kl], agg_sp.at[dstb[kl]], ssem[kl]).wait()

    plsc.subcore_barrier()
    pltpu.sync_copy(agg_sp.at[pl.ds(r0, ROWS_PER_SUB)],
                    out_hbm.at[c].at[pl.ds(r0, ROWS_PER_SUB)])

    @pl.when(s == 15)
    def _():
        pltpu.sync_copy(agg_sp.at[pl.ds(16 * ROWS_PER_SUB, ROWS_TAIL)],
                        out_hbm.at[c].at[pl.ds(16 * ROWS_PER_SUB, ROWS_TAIL)])


@jax.jit
def _sc_aggregate(t, src_e, dst_e, norm_e, zeros):
    mesh = plsc.VectorSubcoreMesh(core_axis_name="c", subcore_axis_name="s")
    cp = pltpu.CompilerParams()
    if "needs_layout_passes" in pltpu.CompilerParams.__dataclass_fields__:
        cp = dataclasses.replace(cp, needs_layout_passes=False)
    kfn = pl.kernel(
        _sc_agg_kernel,
        out_type=jax.ShapeDtypeStruct((2, N_NODES, D), _f32),
        mesh=mesh,
        scratch_types=(
            [pltpu.VMEM((G,), jnp.int32) for _ in range(3)]
            + [pltpu.VMEM((G,), jnp.int32) for _ in range(3)]
            + [pltpu.VMEM((G,), _f32) for _ in range(3)]
            + [pltpu.VMEM((G, D), _f32) for _ in range(3)]
            + [pltpu.VMEM_SHARED((N_NODES, D), _f32)]
            + [pltpu.SemaphoreType.DMA for _ in range(15)]
        ),
        compiler_params=cp,
    )
    return kfn(t, src_e, dst_e, norm_e, zeros)


def _mm_first_kernel(x_ref, w_ref, o_ref):
    o_ref[...] = jnp.dot(x_ref[...], w_ref[...],
                         preferred_element_type=_f32)


def _mm_mid_kernel(p_ref, b_ref, w_ref, o_ref):
    h = jax.nn.relu(p_ref[0] + p_ref[1] + b_ref[...])
    o_ref[...] = jnp.dot(h, w_ref[...], preferred_element_type=_f32)


def _head_kernel(p_ref, b_ref, fcw_ref, fcb_ref, h_ref, y_ref):
    h = p_ref[0] + p_ref[1] + b_ref[...]
    h_ref[...] = h
    y = jnp.dot(h, fcw_ref[...], preferred_element_type=_f32)
    y = y + fcb_ref[...]
    m = jnp.max(y, axis=1, keepdims=True)
    e = jnp.exp(y - m)
    lse = jnp.log(jnp.sum(e, axis=1, keepdims=True)) + m
    y_ref[...] = y - lse


def _mm_first(x, w):
    return pl.pallas_call(
        _mm_first_kernel,
        out_shape=jax.ShapeDtypeStruct((x.shape[0], w.shape[1]), _f32),
    )(x, w)


def _mm_mid(p, b, w):
    return pl.pallas_call(
        _mm_mid_kernel,
        out_shape=jax.ShapeDtypeStruct((p.shape[1], w.shape[1]), _f32),
    )(p, b.reshape(1, -1), w)


def _head(p, b, fcW, fcb):
    return pl.pallas_call(
        _head_kernel,
        out_shape=(jax.ShapeDtypeStruct((p.shape[1], D), _f32),
                   jax.ShapeDtypeStruct((p.shape[1], fcW.shape[1]), _f32)),
    )(p, b.reshape(1, -1), fcW, fcb.reshape(1, -1))


def kernel(x, edge_index, edge_weight, W1, b1, W2, b2, W3, b3, fcW, fcb):
    pad = E_PAD - N_EDGES
    pad_idx = (jnp.arange(pad, dtype=jnp.int32) * 13) % N_NODES
    src_e = jnp.concatenate([edge_index[0], pad_idx])
    dst_e = jnp.concatenate([edge_index[1], pad_idx])
    norm_e = jnp.concatenate([edge_weight, jnp.zeros((pad,), _f32)])
    zeros = jnp.zeros((N_NODES, D), _f32)

    t1 = _mm_first(x, W1)
    p1 = _sc_aggregate(t1, src_e, dst_e, norm_e, zeros)
    t2 = _mm_mid(p1, b1, W2)
    p2 = _sc_aggregate(t2, src_e, dst_e, norm_e, zeros)
    t3 = _mm_mid(p2, b2, W3)
    p3 = _sc_aggregate(t3, src_e, dst_e, norm_e, zeros)
    h, y = _head(p3, b3, fcW, fcb)
    return h, y


# speedup vs baseline: 2.1458x; 1.1780x over previous
import dataclasses
import functools

import jax
import jax.numpy as jnp
from jax import lax
from jax.experimental import pallas as pl
from jax.experimental.pallas import tpu as pltpu
from jax.experimental.pallas import tpu_sc as plsc

N_NODES = 10000
N_EDGES = 320000
D = 128
G = 128
NW = 32
GROUPS_PER_W = 81
E_PAD = NW * GROUPS_PER_W * G
NBUF = 3
N_ITERS = GROUPS_PER_W // NBUF
ROWS_PER_SUB = 624
ROWS_TAIL = N_NODES - 16 * ROWS_PER_SUB

_f32 = jnp.float32


def _sc_agg_kernel(t_hbm, src_hbm, dst_hbm, norm_hbm, zeros_hbm, out_hbm,
                   src0, src1, src2, dst0, dst1, dst2, nrm0, nrm1, nrm2,
                   rows0, rows1, rows2, agg_sp,
                   gs0, gs1, gs2, ss0, ss1, ss2, ds0, ds1, ds2, isem):
    c = lax.axis_index("c")
    s = lax.axis_index("s")
    wid = s * 2 + c
    srcb = (src0, src1, src2)
    dstb = (dst0, dst1, dst2)
    nrmb = (nrm0, nrm1, nrm2)
    rows = (rows0, rows1, rows2)
    gsem = (gs0, gs1, gs2)
    ssem = (ss0, ss1, ss2)
    dsem = (ds0, ds1, ds2)

    e_base = wid * (GROUPS_PER_W * G)

    def src_sl(g):
        return src_hbm.at[pl.ds(e_base + g * G, G)]

    def dst_sl(g):
        return dst_hbm.at[pl.ds(e_base + g * G, G)]

    def nrm_sl(g):
        return norm_hbm.at[pl.ds(e_base + g * G, G)]

    for k in range(2):
        pltpu.sync_copy(src_sl(k), srcb[k])
        pltpu.sync_copy(nrm_sl(k), nrmb[k])
        pltpu.make_async_copy(dst_sl(k), dstb[k], dsem[k]).start()
        pltpu.make_async_copy(t_hbm.at[srcb[k]], rows[k], gsem[k]).start()

    r0 = s * ROWS_PER_SUB
    pltpu.sync_copy(zeros_hbm.at[pl.ds(r0, ROWS_PER_SUB)],
                    agg_sp.at[pl.ds(r0, ROWS_PER_SUB)])

    @pl.when(s == 15)
    def _():
        pltpu.sync_copy(zeros_hbm.at[pl.ds(16 * ROWS_PER_SUB, ROWS_TAIL)],
                        agg_sp.at[pl.ds(16 * ROWS_PER_SUB, ROWS_TAIL)])

    plsc.subcore_barrier()

    def _scale(buf, nbuf):
        def tile_body(r16, carry):
            for dr in range(8):
                r = r16 + dr
                nb = plsc.load_gather(nbuf, [lax.broadcast(r, (16,))])
                for j in range(8):
                    sl = pl.ds(j * 16, 16)
                    buf[r, sl] = buf[r, sl] * nb
            return carry

        lax.fori_loop(0, G // 8, lambda t, cy: tile_body(t * 8, cy), 0)

    def iter_body(i, carry):
        for k in range(NBUF):
            g = i * NBUF + k
            bp = (k + 2) % NBUF
            gp2 = g + 2
            have_next = gp2 <= GROUPS_PER_W - 1

            @pl.when(have_next)
            def _():
                pltpu.make_async_copy(src_sl(gp2), srcb[bp], isem).start()
                pltpu.make_async_copy(nrm_sl(gp2), nrmb[bp], isem).start()

            pltpu.make_async_copy(t_hbm.at[srcb[k]], rows[k],
                                  gsem[k]).wait()
            _scale(rows[k], nrmb[k])
            pltpu.make_async_copy(dst_sl(g), dstb[k], dsem[k]).wait()
            pltpu.make_async_copy(rows[k], agg_sp.at[dstb[k]],
                                  ssem[k]).start(add=True)

            @pl.when(g >= 1)
            def _():
                pltpu.make_async_copy(rows[bp], agg_sp.at[dstb[bp]],
                                      ssem[bp]).wait()

            @pl.when(have_next)
            def _():
                pltpu.make_async_copy(dst_sl(gp2), dstb[bp],
                                      dsem[bp]).start()
                pltpu.make_async_copy(src_sl(gp2), srcb[bp], isem).wait()
                pltpu.make_async_copy(nrm_sl(gp2), nrmb[bp], isem).wait()
                pltpu.make_async_copy(t_hbm.at[srcb[bp]], rows[bp],
                                      gsem[bp]).start()
        return carry

    lax.fori_loop(0, N_ITERS, iter_body, 0)

    kl = (GROUPS_PER_W - 1) % NBUF
    pltpu.make_async_copy(rows[kl], agg_sp.at[dstb[kl]], ssem[kl]).wait()

    plsc.subcore_barrier()
    pltpu.sync_copy(agg_sp.at[pl.ds(r0, ROWS_PER_SUB)],
                    out_hbm.at[c].at[pl.ds(r0, ROWS_PER_SUB)])

    @pl.when(s == 15)
    def _():
        pltpu.sync_copy(agg_sp.at[pl.ds(16 * ROWS_PER_SUB, ROWS_TAIL)],
                        out_hbm.at[c].at[pl.ds(16 * ROWS_PER_SUB, ROWS_TAIL)])


@jax.jit
def _sc_aggregate(t, src_e, dst_e, norm_e, zeros):
    mesh = plsc.VectorSubcoreMesh(core_axis_name="c", subcore_axis_name="s")
    cp = pltpu.CompilerParams()
    if "needs_layout_passes" in pltpu.CompilerParams.__dataclass_fields__:
        cp = dataclasses.replace(cp, needs_layout_passes=False)
    kfn = pl.kernel(
        _sc_agg_kernel,
        out_type=jax.ShapeDtypeStruct((2, N_NODES, D), _f32),
        mesh=mesh,
        scratch_types=(
            [pltpu.VMEM((G,), jnp.int32) for _ in range(3)]
            + [pltpu.VMEM((G,), jnp.int32) for _ in range(3)]
            + [pltpu.VMEM((G,), _f32) for _ in range(3)]
            + [pltpu.VMEM((G, D), _f32) for _ in range(3)]
            + [pltpu.VMEM_SHARED((N_NODES, D), _f32)]
            + [pltpu.SemaphoreType.DMA for _ in range(10)]
        ),
        compiler_params=cp,
    )
    return kfn(t, src_e, dst_e, norm_e, zeros)


def _mm_first_kernel(x_ref, w_ref, o_ref):
    o_ref[...] = jnp.dot(x_ref[...], w_ref[...],
                         preferred_element_type=_f32)


def _mm_mid_kernel(p_ref, b_ref, w_ref, o_ref):
    h = jax.nn.relu(p_ref[0] + p_ref[1] + b_ref[...])
    o_ref[...] = jnp.dot(h, w_ref[...], preferred_element_type=_f32)


def _head_kernel(p_ref, b_ref, fcw_ref, fcb_ref, h_ref, y_ref):
    h = p_ref[0] + p_ref[1] + b_ref[...]
    h_ref[...] = h
    y = jnp.dot(h, fcw_ref[...], preferred_element_type=_f32)
    y = y + fcb_ref[...]
    m = jnp.max(y, axis=1, keepdims=True)
    e = jnp.exp(y - m)
    lse = jnp.log(jnp.sum(e, axis=1, keepdims=True)) + m
    y_ref[...] = y - lse


def _mm_first(x, w):
    return pl.pallas_call(
        _mm_first_kernel,
        out_shape=jax.ShapeDtypeStruct((x.shape[0], w.shape[1]), _f32),
    )(x, w)


def _mm_mid(p, b, w):
    return pl.pallas_call(
        _mm_mid_kernel,
        out_shape=jax.ShapeDtypeStruct((p.shape[1], w.shape[1]), _f32),
    )(p, b.reshape(1, -1), w)


def _head(p, b, fcW, fcb):
    return pl.pallas_call(
        _head_kernel,
        out_shape=(jax.ShapeDtypeStruct((p.shape[1], D), _f32),
                   jax.ShapeDtypeStruct((p.shape[1], fcW.shape[1]), _f32)),
    )(p, b.reshape(1, -1), fcW, fcb.reshape(1, -1))


def kernel(x, edge_index, edge_weight, W1, b1, W2, b2, W3, b3, fcW, fcb):
    pad = E_PAD - N_EDGES
    pad_idx = (jnp.arange(pad, dtype=jnp.int32) * 13) % N_NODES
    src_e = jnp.concatenate([edge_index[0], pad_idx])
    dst_e = jnp.concatenate([edge_index[1], pad_idx])
    norm_e = jnp.concatenate([edge_weight, jnp.zeros((pad,), _f32)])
    zeros = jnp.zeros((N_NODES, D), _f32)

    t1 = _mm_first(x, W1)
    p1 = _sc_aggregate(t1, src_e, dst_e, norm_e, zeros)
    t2 = _mm_mid(p1, b1, W2)
    p2 = _sc_aggregate(t2, src_e, dst_e, norm_e, zeros)
    t3 = _mm_mid(p2, b2, W3)
    p3 = _sc_aggregate(t3, src_e, dst_e, norm_e, zeros)
    h, y = _head(p3, b3, fcW, fcb)
    return h, y


# speedup vs baseline: 2.1683x; 1.0105x over previous
import dataclasses
import functools

import jax
import jax.numpy as jnp
from jax import lax
from jax.experimental import pallas as pl
from jax.experimental.pallas import tpu as pltpu
from jax.experimental.pallas import tpu_sc as plsc

N_NODES = 10000
N_EDGES = 320000
D = 128
G = 128
NW = 32
GROUPS_PER_W = 81
E_PAD = NW * GROUPS_PER_W * G
NBUF = 3
N_ITERS = GROUPS_PER_W // NBUF
ROWS_PER_SUB = 624
ROWS_TAIL = N_NODES - 16 * ROWS_PER_SUB

_f32 = jnp.float32


def _sc_agg_kernel(t_hbm, src_hbm, dst_hbm, norm_hbm, zeros_hbm, out_hbm,
                   src0, src1, src2, dst0, dst1, dst2, nrm0, nrm1, nrm2,
                   rows0, rows1, rows2, agg_sp,
                   gs0, gs1, gs2, ss0, ss1, ss2, ds0, ds1, ds2, isem):
    c = lax.axis_index("c")
    s = lax.axis_index("s")
    wid = s * 2 + c
    srcb = (src0, src1, src2)
    dstb = (dst0, dst1, dst2)
    nrmb = (nrm0, nrm1, nrm2)
    rows = (rows0, rows1, rows2)
    gsem = (gs0, gs1, gs2)
    ssem = (ss0, ss1, ss2)
    dsem = (ds0, ds1, ds2)

    e_base = wid * (GROUPS_PER_W * G)

    def src_sl(g):
        return src_hbm.at[pl.ds(e_base + g * G, G)]

    def dst_sl(g):
        return dst_hbm.at[pl.ds(e_base + g * G, G)]

    def nrm_sl(g):
        return norm_hbm.at[pl.ds(e_base + g * G, G)]

    for k in range(2):
        pltpu.sync_copy(src_sl(k), srcb[k])
        pltpu.sync_copy(nrm_sl(k), nrmb[k])
        pltpu.make_async_copy(dst_sl(k), dstb[k], dsem[k]).start()
        pltpu.make_async_copy(t_hbm.at[srcb[k]], rows[k], gsem[k]).start()

    r0 = s * ROWS_PER_SUB
    pltpu.sync_copy(zeros_hbm.at[pl.ds(r0, ROWS_PER_SUB)],
                    agg_sp.at[pl.ds(r0, ROWS_PER_SUB)])

    @pl.when(s == 15)
    def _():
        pltpu.sync_copy(zeros_hbm.at[pl.ds(16 * ROWS_PER_SUB, ROWS_TAIL)],
                        agg_sp.at[pl.ds(16 * ROWS_PER_SUB, ROWS_TAIL)])

    plsc.subcore_barrier()

    def _scale(buf, nbuf):
        def tile_body(r16, carry):
            for dr in range(4):
                r = r16 + dr
                nb = plsc.load_gather(nbuf, [lax.broadcast(r, (16,))])
                for j in range(8):
                    sl = pl.ds(j * 16, 16)
                    buf[r, sl] = buf[r, sl] * nb
            return carry

        lax.fori_loop(0, G // 4, lambda t, cy: tile_body(t * 4, cy), 0)

    def iter_body(i, carry):
        for k in range(NBUF):
            g = i * NBUF + k
            bp = (k + 2) % NBUF
            gp2 = g + 2
            have_next = gp2 <= GROUPS_PER_W - 1

            @pl.when(have_next)
            def _():
                pltpu.make_async_copy(src_sl(gp2), srcb[bp], isem).start()
                pltpu.make_async_copy(nrm_sl(gp2), nrmb[bp], isem).start()

            pltpu.make_async_copy(t_hbm.at[srcb[k]], rows[k],
                                  gsem[k]).wait()
            _scale(rows[k], nrmb[k])
            pltpu.make_async_copy(dst_sl(g), dstb[k], dsem[k]).wait()
            pltpu.make_async_copy(rows[k], agg_sp.at[dstb[k]],
                                  ssem[k]).start(add=True)

            @pl.when(g >= 1)
            def _():
                pltpu.make_async_copy(rows[bp], agg_sp.at[dstb[bp]],
                                      ssem[bp]).wait()

            @pl.when(have_next)
            def _():
                pltpu.make_async_copy(dst_sl(gp2), dstb[bp],
                                      dsem[bp]).start()
                pltpu.make_async_copy(src_sl(gp2), srcb[bp], isem).wait()
                pltpu.make_async_copy(nrm_sl(gp2), nrmb[bp], isem).wait()
                pltpu.make_async_copy(t_hbm.at[srcb[bp]], rows[bp],
                                      gsem[bp]).start()
        return carry

    lax.fori_loop(0, N_ITERS, iter_body, 0)

    kl = (GROUPS_PER_W - 1) % NBUF
    pltpu.make_async_copy(rows[kl], agg_sp.at[dstb[kl]], ssem[kl]).wait()

    plsc.subcore_barrier()
    pltpu.sync_copy(agg_sp.at[pl.ds(r0, ROWS_PER_SUB)],
                    out_hbm.at[c].at[pl.ds(r0, ROWS_PER_SUB)])

    @pl.when(s == 15)
    def _():
        pltpu.sync_copy(agg_sp.at[pl.ds(16 * ROWS_PER_SUB, ROWS_TAIL)],
                        out_hbm.at[c].at[pl.ds(16 * ROWS_PER_SUB, ROWS_TAIL)])


@jax.jit
def _sc_aggregate(t, src_e, dst_e, norm_e, zeros):
    mesh = plsc.VectorSubcoreMesh(core_axis_name="c", subcore_axis_name="s")
    cp = pltpu.CompilerParams()
    if "needs_layout_passes" in pltpu.CompilerParams.__dataclass_fields__:
        cp = dataclasses.replace(cp, needs_layout_passes=False)
    kfn = pl.kernel(
        _sc_agg_kernel,
        out_type=jax.ShapeDtypeStruct((2, N_NODES, D), _f32),
        mesh=mesh,
        scratch_types=(
            [pltpu.VMEM((G,), jnp.int32) for _ in range(3)]
            + [pltpu.VMEM((G,), jnp.int32) for _ in range(3)]
            + [pltpu.VMEM((G,), _f32) for _ in range(3)]
            + [pltpu.VMEM((G, D), _f32) for _ in range(3)]
            + [pltpu.VMEM_SHARED((N_NODES, D), _f32)]
            + [pltpu.SemaphoreType.DMA for _ in range(10)]
        ),
        compiler_params=cp,
    )
    return kfn(t, src_e, dst_e, norm_e, zeros)


def _mm_first_kernel(x_ref, w_ref, o_ref):
    o_ref[...] = jnp.dot(x_ref[...], w_ref[...],
                         preferred_element_type=_f32)


def _mm_mid_kernel(p_ref, b_ref, w_ref, o_ref):
    h = jax.nn.relu(p_ref[0] + p_ref[1] + b_ref[...])
    o_ref[...] = jnp.dot(h, w_ref[...], preferred_element_type=_f32)


def _head_kernel(p_ref, b_ref, fcw_ref, fcb_ref, h_ref, y_ref):
    h = p_ref[0] + p_ref[1] + b_ref[...]
    h_ref[...] = h
    y = jnp.dot(h, fcw_ref[...], preferred_element_type=_f32)
    y = y + fcb_ref[...]
    m = jnp.max(y, axis=1, keepdims=True)
    e = jnp.exp(y - m)
    lse = jnp.log(jnp.sum(e, axis=1, keepdims=True)) + m
    y_ref[...] = y - lse


def _mm_first(x, w):
    return pl.pallas_call(
        _mm_first_kernel,
        out_shape=jax.ShapeDtypeStruct((x.shape[0], w.shape[1]), _f32),
    )(x, w)


def _mm_mid(p, b, w):
    return pl.pallas_call(
        _mm_mid_kernel,
        out_shape=jax.ShapeDtypeStruct((p.shape[1], w.shape[1]), _f32),
    )(p, b.reshape(1, -1), w)


def _head(p, b, fcW, fcb):
    return pl.pallas_call(
        _head_kernel,
        out_shape=(jax.ShapeDtypeStruct((p.shape[1], D), _f32),
                   jax.ShapeDtypeStruct((p.shape[1], fcW.shape[1]), _f32)),
    )(p, b.reshape(1, -1), fcW, fcb.reshape(1, -1))


def kernel(x, edge_index, edge_weight, W1, b1, W2, b2, W3, b3, fcW, fcb):
    pad = E_PAD - N_EDGES
    pad_idx = (jnp.arange(pad, dtype=jnp.int32) * 13) % N_NODES
    src_e = jnp.concatenate([edge_index[0], pad_idx])
    dst_e = jnp.concatenate([edge_index[1], pad_idx])
    norm_e = jnp.concatenate([edge_weight, jnp.zeros((pad,), _f32)])
    zeros = jnp.zeros((N_NODES, D), _f32)

    t1 = _mm_first(x, W1)
    p1 = _sc_aggregate(t1, src_e, dst_e, norm_e, zeros)
    t2 = _mm_mid(p1, b1, W2)
    p2 = _sc_aggregate(t2, src_e, dst_e, norm_e, zeros)
    t3 = _mm_mid(p2, b2, W3)
    p3 = _sc_aggregate(t3, src_e, dst_e, norm_e, zeros)
    h, y = _head(p3, b3, fcW, fcb)
    return h, y
